# baseline (speedup 1.0000x reference)
import functools

import jax
import jax.numpy as jnp
from jax import lax
from jax.experimental import pallas as pl
from jax.experimental.pallas import tpu as pltpu
from jax.experimental.pallas import tpu_sc as plsc

_NC = 2
_NS = 16
_NW = _NC * _NS
_RPC = 8


def _make_gather(v, d, batch, hist, nslots):
    assert batch % (_NW * _RPC) == 0
    b_per_w = batch // _NW
    n_chunks = b_per_w // _RPC
    assert n_chunks % nslots == 0 and n_chunks >= 2 * nslots
    mesh = plsc.VectorSubcoreMesh(core_axis_name="c", subcore_axis_name="s",
                                  num_cores=_NC, num_subcores=_NS)
    scratch = (
        [pltpu.VMEM((_RPC, hist), jnp.int32) for _ in range(nslots)]
        + [pltpu.VMEM((_RPC, hist, d), jnp.float32) for _ in range(nslots)]
        + [pltpu.SemaphoreType.DMA for _ in range(2 * nslots)]
    )

    @functools.partial(
        pl.kernel,
        mesh=mesh,
        out_type=jax.ShapeDtypeStruct((batch, hist, d), jnp.float32),
        scratch_types=scratch,
        compiler_params=pltpu.CompilerParams(use_tc_tiling_on_sc=False),
    )
    def gather_kernel(x_hbm, table_hbm, out_hbm, *refs):
        idx_v = list(refs[0:nslots])
        rows_v = list(refs[nslots:2 * nslots])
        gsem = list(refs[2 * nslots:3 * nslots])
        osem = list(refs[3 * nslots:4 * nslots])
        wid = lax.axis_index("s") * _NC + lax.axis_index("c")
        base = wid * b_per_w

        def load_idx(i, slot):
            pltpu.sync_copy(x_hbm.at[pl.ds(base + i * _RPC, _RPC), :],
                            idx_v[slot])

        def fire_gather(slot):
            for r in range(_RPC):
                pltpu.async_copy(table_hbm.at[idx_v[slot].at[r]],
                                 rows_v[slot].at[r], gsem[slot])

        def wait_gather(slot):
            for r in range(_RPC):
                pltpu.make_async_copy(table_hbm.at[idx_v[slot].at[r]],
                                      rows_v[slot].at[r],
                                      gsem[slot]).wait()

        def fire_store(i, slot):
            pltpu.async_copy(rows_v[slot],
                             out_hbm.at[pl.ds(base + i * _RPC, _RPC), :, :],
                             osem[slot])

        def wait_store(i, slot):
            pltpu.make_async_copy(rows_v[slot],
                                  out_hbm.at[pl.ds(base + i * _RPC, _RPC),
                                             :, :],
                                  osem[slot]).wait()

        for j in range(nslots - 1):
            load_idx(j, j)
            fire_gather(j)

        load_idx(nslots - 1, nslots - 1)
        fire_gather(nslots - 1)
        wait_gather(0)
        fire_store(0, 0)

        @pl.loop(0, (n_chunks - nslots) // nslots)
        def _(ii):
            for b in range(nslots):
                i = 1 + nslots * ii + b
                s = (1 + b) % nslots
                p = b % nslots
                load_idx(i + nslots - 1, p)
                wait_store(i - 1, p)
                fire_gather(p)
                wait_gather(s)
                fire_store(i, s)

        for i in range(n_chunks - nslots + 1, n_chunks):
            s = i % nslots
            wait_gather(s)
            fire_store(i, s)

        for i in range(n_chunks - nslots, n_chunks):
            wait_store(i, i % nslots)

    return gather_kernel


@jax.jit
def kernel(x, table):
    batch, hist = x.shape
    vocab, dim = table.shape
    return _make_gather(vocab, dim, batch, hist, 4)(x, table)


# speedup vs baseline: 1.0577x; 1.0577x over previous
import functools

import jax
import jax.numpy as jnp
from jax import lax
from jax.experimental import pallas as pl
from jax.experimental.pallas import tpu as pltpu
from jax.experimental.pallas import tpu_sc as plsc

_NC = 2
_NS = 16
_NW = _NC * _NS


def _make_gather(v, d, hist, batch):
    bpw = batch // _NW
    assert bpw * _NW == batch and hist % 2 == 0
    mesh = plsc.VectorSubcoreMesh(core_axis_name="c", subcore_axis_name="s",
                                  num_cores=_NC, num_subcores=_NS)

    @functools.partial(
        pl.kernel,
        mesh=mesh,
        out_type=jax.ShapeDtypeStruct((hist, batch, d), jnp.float32),
        compiler_params=pltpu.CompilerParams(use_tc_tiling_on_sc=False),
        scratch_types=[
            pltpu.VMEM((bpw,), jnp.int32),
            pltpu.VMEM((bpw,), jnp.int32),
            pltpu.VMEM((bpw, d), jnp.float32),
            pltpu.VMEM((bpw, d), jnp.float32),
            pltpu.SemaphoreType.DMA,
            pltpu.SemaphoreType.DMA,
            pltpu.SemaphoreType.DMA,
            pltpu.SemaphoreType.DMA,
        ],
    )
    def gather_kernel(tab_hbm, xt_hbm, out_hbm, i0, i1, r0, r1,
                      g0, g1, o0, o1):
        idx_v = [i0, i1]
        rows = [r0, r1]
        gsem = [g0, g1]
        osem = [o0, o1]
        w = lax.axis_index("s") * _NC + lax.axis_index("c")
        b_base = w * bpw

        def prep(h, slot):
            pltpu.sync_copy(xt_hbm.at[h, pl.ds(b_base, bpw)], idx_v[slot])

        def fire_gather(slot):
            pltpu.async_copy(tab_hbm.at[idx_v[slot]], rows[slot],
                             gsem[slot])

        def wait_gather(slot):
            pltpu.make_async_copy(tab_hbm.at[idx_v[slot]], rows[slot],
                                  gsem[slot]).wait()

        def fire_out(h, slot):
            pltpu.async_copy(rows[slot],
                             out_hbm.at[h, pl.ds(b_base, bpw), :],
                             osem[slot])

        def wait_out(h, slot):
            pltpu.make_async_copy(rows[slot],
                                  out_hbm.at[h, pl.ds(b_base, bpw), :],
                                  osem[slot]).wait()

        prep(0, 0)
        fire_gather(0)

        @pl.loop(0, hist // 2)
        def _(hh):
            for p in (0, 1):
                h = 2 * hh + p
                s = p
                o = 1 - p

                @pl.when(h + 1 < hist)
                def _(h=h, s=s, o=o):
                    prep(h + 1, o)

                    @pl.when(h >= 1)
                    def _():
                        wait_out(h - 1, o)

                    fire_gather(o)

                wait_gather(s)
                fire_out(h, s)

        wait_out(hist - 2, 0)
        wait_out(hist - 1, 1)

    return gather_kernel


@jax.jit
def kernel(x, table):
    batch, hist = x.shape
    vocab, dim = table.shape
    xt = x.T
    out_hm = _make_gather(vocab, dim, hist, batch)(table, xt)
    return jnp.transpose(out_hm, (1, 0, 2))
